# baseline (speedup 1.0000x reference)
import functools

import jax
import jax.numpy as jnp
from jax.experimental import pallas as pl


def _pick_bm(n):
    for bm in (400, 200, 100, 50, 25, 8, 4, 2, 1):
        if n % bm == 0:
            return bm
    return n


def _linear_kernel(x_ref, w_ref, b_ref, out_ref):
    out_ref[...] = (
        jnp.dot(x_ref[...], w_ref[...], preferred_element_type=jnp.float32)
        + b_ref[...]
    )


def _agg_linear_kernel(adj_ref, z_ref, w_ref, b_ref, out_ref):
    h = jnp.dot(adj_ref[...], z_ref[...], preferred_element_type=jnp.float32)
    h = jnp.maximum(h, 0.0)
    out_ref[...] = (
        jnp.dot(h, w_ref[...], preferred_element_type=jnp.float32) + b_ref[...]
    )


def _agg_kernel(adj_ref, z_ref, out_ref):
    out_ref[...] = jnp.dot(
        adj_ref[...], z_ref[...], preferred_element_type=jnp.float32
    )


@jax.jit
def kernel(x, Adj, W1, b1, W2, b2):
    n, d_in = x.shape
    d_h = W1.shape[1]
    d_out = W2.shape[1]
    b1r = b1.reshape(1, d_h)
    b2r = b2.reshape(1, d_out)

    bm = _pick_bm(n)
    grid = (n // bm,)

    z1 = pl.pallas_call(
        _linear_kernel,
        grid=grid,
        in_specs=[
            pl.BlockSpec((bm, d_in), lambda i: (i, 0)),
            pl.BlockSpec((d_in, d_h), lambda i: (0, 0)),
            pl.BlockSpec((1, d_h), lambda i: (0, 0)),
        ],
        out_specs=pl.BlockSpec((bm, d_h), lambda i: (i, 0)),
        out_shape=jax.ShapeDtypeStruct((n, d_h), jnp.float32),
    )(x, W1, b1r)

    z2 = pl.pallas_call(
        _agg_linear_kernel,
        grid=grid,
        in_specs=[
            pl.BlockSpec((bm, n), lambda i: (i, 0)),
            pl.BlockSpec((n, d_h), lambda i: (0, 0)),
            pl.BlockSpec((d_h, d_out), lambda i: (0, 0)),
            pl.BlockSpec((1, d_out), lambda i: (0, 0)),
        ],
        out_specs=pl.BlockSpec((bm, d_out), lambda i: (i, 0)),
        out_shape=jax.ShapeDtypeStruct((n, d_out), jnp.float32),
    )(Adj, z1, W2, b2r)

    out = pl.pallas_call(
        _agg_kernel,
        grid=grid,
        in_specs=[
            pl.BlockSpec((bm, n), lambda i: (i, 0)),
            pl.BlockSpec((n, d_out), lambda i: (0, 0)),
        ],
        out_specs=pl.BlockSpec((bm, d_out), lambda i: (i, 0)),
        out_shape=jax.ShapeDtypeStruct((n, d_out), jnp.float32),
    )(Adj, z2)

    return out


# speedup vs baseline: 1.0985x; 1.0985x over previous
import functools

import jax
import jax.numpy as jnp
from jax.experimental import pallas as pl
from jax.experimental.pallas import tpu as pltpu


def _pick_bm(n):
    for bm in (400, 200, 100, 50, 25, 8, 4, 2, 1):
        if n % bm == 0:
            return bm
    return n


def _gcn_kernel(adj_ref, x_ref, w1_ref, b1_ref, w2_ref, b2_ref,
                out_ref, z1_s, z2_s, *, bm, gsteps):
    i = pl.program_id(0)

    @pl.when(i == 0)
    def _():
        z1_s[...] = (
            jnp.dot(x_ref[...], w1_ref[...], preferred_element_type=jnp.float32)
            + b1_ref[...]
        )

    @pl.when(i < gsteps)
    def _():
        h = jnp.dot(adj_ref[...], z1_s[...], preferred_element_type=jnp.float32)
        h = jnp.maximum(h, 0.0)
        z2 = (
            jnp.dot(h, w2_ref[...], preferred_element_type=jnp.float32)
            + b2_ref[...]
        )
        z2_s[pl.ds(i * bm, bm), :] = z2
        out_ref[...] = z2

    @pl.when(i >= gsteps)
    def _():
        out_ref[...] = jnp.dot(
            adj_ref[...], z2_s[...], preferred_element_type=jnp.float32
        )


@jax.jit
def kernel(x, Adj, W1, b1, W2, b2):
    n, d_in = x.shape
    d_h = W1.shape[1]
    d_out = W2.shape[1]
    b1r = b1.reshape(1, d_h)
    b2r = b2.reshape(1, d_out)

    bm = _pick_bm(n)
    g = n // bm

    body = functools.partial(_gcn_kernel, bm=bm, gsteps=g)

    out = pl.pallas_call(
        body,
        grid=(2 * g,),
        in_specs=[
            pl.BlockSpec((bm, n), lambda i: (i % g, 0)),
            pl.BlockSpec((n, d_in), lambda i: (0, 0)),
            pl.BlockSpec((d_in, d_h), lambda i: (0, 0)),
            pl.BlockSpec((1, d_h), lambda i: (0, 0)),
            pl.BlockSpec((d_h, d_out), lambda i: (0, 0)),
            pl.BlockSpec((1, d_out), lambda i: (0, 0)),
        ],
        out_specs=pl.BlockSpec((bm, d_out), lambda i: (i % g, 0)),
        out_shape=jax.ShapeDtypeStruct((n, d_out), jnp.float32),
        scratch_shapes=[
            pltpu.VMEM((n, d_h), jnp.float32),
            pltpu.VMEM((n, d_out), jnp.float32),
        ],
    )(Adj, x, W1, b1r, W2, b2r)

    return out


# speedup vs baseline: 1.1022x; 1.0034x over previous
import functools

import jax
import jax.numpy as jnp
from jax.experimental import pallas as pl
from jax.experimental.pallas import tpu as pltpu


def _pick_bm(n):
    for bm in (400, 200, 100, 50, 25, 8, 4, 2, 1):
        if n % bm == 0:
            return bm
    return n


def _gcn_kernel(adj_ref, x_ref, w1_ref, b1_ref, w2_ref, b2_ref,
                out_ref, z1_s, z2_s, *, bm, gsteps):
    i = pl.program_id(0)

    @pl.when(i == 0)
    def _():
        z1_s[...] = (
            jnp.dot(x_ref[...], w1_ref[...], preferred_element_type=jnp.float32)
            + b1_ref[...]
        )

    @pl.when(i < gsteps)
    def _():
        h = jnp.dot(adj_ref[...], z1_s[...], preferred_element_type=jnp.float32)
        h = jnp.maximum(h, 0.0)
        z2 = (
            jnp.dot(h, w2_ref[...], preferred_element_type=jnp.float32)
            + b2_ref[...]
        )
        z2_s[pl.ds(i * bm, bm), :] = z2

    @pl.when(i >= gsteps)
    def _():
        out_ref[...] = jnp.dot(
            adj_ref[...], z2_s[...], preferred_element_type=jnp.float32
        )


@jax.jit
def kernel(x, Adj, W1, b1, W2, b2):
    n, d_in = x.shape
    d_h = W1.shape[1]
    d_out = W2.shape[1]
    b1r = b1.reshape(1, d_h)
    b2r = b2.reshape(1, d_out)

    bm = _pick_bm(n)
    g = n // bm

    body = functools.partial(_gcn_kernel, bm=bm, gsteps=g)

    out = pl.pallas_call(
        body,
        grid=(2 * g,),
        in_specs=[
            pl.BlockSpec((bm, n), lambda i: (i % g, 0)),
            pl.BlockSpec((n, d_in), lambda i: (0, 0)),
            pl.BlockSpec((d_in, d_h), lambda i: (0, 0)),
            pl.BlockSpec((1, d_h), lambda i: (0, 0)),
            pl.BlockSpec((d_h, d_out), lambda i: (0, 0)),
            pl.BlockSpec((1, d_out), lambda i: (0, 0)),
        ],
        out_specs=pl.BlockSpec(
            (bm, d_out), lambda i: (jnp.where(i < g, 0, i - g), 0)
        ),
        out_shape=jax.ShapeDtypeStruct((n, d_out), jnp.float32),
        scratch_shapes=[
            pltpu.VMEM((n, d_h), jnp.float32),
            pltpu.VMEM((n, d_out), jnp.float32),
        ],
    )(Adj, x, W1, b1r, W2, b2r)

    return out
